# baseline (speedup 1.0000x reference)
import functools

import jax
import jax.numpy as jnp
import numpy as np
from jax import lax
from jax.experimental import pallas as pl
from jax.experimental.pallas import tpu as pltpu
from jax.experimental.pallas import tpu_sc as plsc

N = 10000
E2 = 320000
A = 2500
G = 32
D = 128
RBF = 16
SBF = 112
H = 8
HC = D // H
L = 3
EPS = 1e-8

_NC = 2
_NS = 16
_NW = _NC * _NS
_C = 128
_K = 79
_EPW = _C * _K
_E2P = _NW * _EPW
_PAD = _E2P - E2
_NP = 10240
_RPT = _NP // _NS
_CB = 64
_KB = _EPW // _CB


def _lin(p, x):
    y = x @ p["W"]
    if "b" in p:
        y = y + p["b"]
    return y


def _silu(x):
    return x * jax.nn.sigmoid(x)


_FILT_BLK = 2048


def _filt_body(sbf_ref, w_ref, b_ref, o_ref):
    o_ref[...] = (
        jnp.dot(sbf_ref[...], w_ref[...], preferred_element_type=jnp.float32)
        + b_ref[...]
    )


def _filt_matmul(sbf, w, b):
    grid = (_E2P // _FILT_BLK,)
    return pl.pallas_call(
        _filt_body,
        grid=grid,
        in_specs=[
            pl.BlockSpec((_FILT_BLK, SBF), lambda i: (i, 0)),
            pl.BlockSpec((SBF, D), lambda i: (0, 0)),
            pl.BlockSpec((1, D), lambda i: (0, 0)),
        ],
        out_specs=pl.BlockSpec((_FILT_BLK, D), lambda i: (i, 0)),
        out_shape=jax.ShapeDtypeStruct((_E2P, D), jnp.float32),
    )(sbf, w, b.reshape(1, D))


def _edge_scores_body(qx, kx, eat, src, dst, attr, s_out,
                      dstb, srcb, attrb, qb, kb, eb, sb, sem):
    cid = lax.axis_index("c")
    sid = lax.axis_index("s")
    wid = sid * _NC + cid
    base0 = wid * _EPW
    iota = lax.iota(jnp.int32, 16)

    def chunk(i, carry):
        base = base0 + i * _C
        pltpu.sync_copy(dst.at[pl.ds(base, _C)], dstb)
        pltpu.sync_copy(src.at[pl.ds(base, _C)], srcb)
        pltpu.sync_copy(attr.at[pl.ds(base, _C)], attrb)
        pltpu.async_copy(qx.at[dstb], qb, sem).wait()
        pltpu.async_copy(kx.at[srcb], kb, sem).wait()
        pltpu.async_copy(eat.at[attrb], eb, sem).wait()

        def group(g, c2):
            ridx = g * 16 + iota
            for h in range(H):
                acc = jnp.zeros((16,), jnp.float32)
                for cc in range(HC):
                    col = jnp.full((16,), h * HC + cc, jnp.int32)
                    qv = plsc.load_gather(qb, [ridx, col])
                    kv = plsc.load_gather(kb, [ridx, col])
                    ev = plsc.load_gather(eb, [ridx, col])
                    acc = acc + qv * (kv + ev)
                hcol = jnp.full((16,), h, jnp.int32)
                plsc.store_scatter(sb, [ridx, hcol], acc * 0.25)
            return c2

        lax.fori_loop(0, _C // 16, group, 0)
        pltpu.sync_copy(sb, s_out.at[pl.ds(base, _C)])
        return carry

    lax.fori_loop(0, _K, chunk, 0)


_edge_scores = pl.kernel(
    _edge_scores_body,
    out_type=jax.ShapeDtypeStruct((_E2P, H), jnp.float32),
    mesh=plsc.VectorSubcoreMesh(
        core_axis_name="c", subcore_axis_name="s",
        num_cores=_NC, num_subcores=_NS,
    ),
    compiler_params=pltpu.CompilerParams(needs_layout_passes=False),
    scratch_types=[
        pltpu.VMEM((_C,), jnp.int32),
        pltpu.VMEM((_C,), jnp.int32),
        pltpu.VMEM((_C,), jnp.int32),
        pltpu.VMEM((_C, D), jnp.float32),
        pltpu.VMEM((_C, D), jnp.float32),
        pltpu.VMEM((_C, D), jnp.float32),
        pltpu.VMEM((_C, H), jnp.float32),
        pltpu.SemaphoreType.DMA,
    ],
)


def _edge_msgs_body(vx, eat, filt, w, src, dst, attr, acc_out, wsum_out,
                    dstb, srcb, attrb, vb, eb, fb, wb, accsh, wsumsh, sem):
    cid = lax.axis_index("c")
    sid = lax.axis_index("s")
    wid = sid * _NC + cid
    base0 = wid * _EPW
    iota = lax.iota(jnp.int32, 16)
    zero16 = jnp.zeros((16,), jnp.float32)

    def zrow(r, carry):
        for h in range(H):
            fb[r, pl.ds(h * 16, 16)] = zero16
        wb[r, :] = zero16
        return carry

    lax.fori_loop(0, _CB, zrow, 0)
    zb = sid * _RPT
    for j in range(_RPT // _CB):
        pltpu.sync_copy(fb, accsh.at[pl.ds(zb + j * _CB, _CB)])
        pltpu.sync_copy(wb, wsumsh.at[pl.ds(zb + j * _CB, _CB)])
    plsc.subcore_barrier()

    def chunk(i, carry):
        base = base0 + i * _CB
        pltpu.sync_copy(dst.at[pl.ds(base, _CB)], dstb)
        pltpu.sync_copy(src.at[pl.ds(base, _CB)], srcb)
        pltpu.sync_copy(attr.at[pl.ds(base, _CB)], attrb)
        pltpu.async_copy(vx.at[srcb], vb, sem).wait()
        pltpu.async_copy(eat.at[attrb], eb, sem).wait()
        pltpu.sync_copy(filt.at[pl.ds(base, _CB)], fb)
        pltpu.sync_copy(w.at[pl.ds(base, _CB)], wb)

        def group(g, c2):
            ridx = g * 16 + iota
            for h in range(H):
                hcol = jnp.full((16,), h, jnp.int32)
                wv = plsc.load_gather(wb, [ridx, hcol])
                for cc in range(HC):
                    col = jnp.full((16,), h * HC + cc, jnp.int32)
                    vv = plsc.load_gather(vb, [ridx, col])
                    ev = plsc.load_gather(eb, [ridx, col])
                    fv = plsc.load_gather(fb, [ridx, col])
                    plsc.store_scatter(vb, [ridx, col], (vv + ev) * fv * wv)
            return c2

        lax.fori_loop(0, _CB // 16, group, 0)
        pltpu.sync_copy(vb, accsh.at[dstb], add=True)
        pltpu.sync_copy(wb, wsumsh.at[dstb], add=True)
        return carry

    lax.fori_loop(0, _KB, chunk, 0)
    plsc.subcore_barrier()

    for j in range(_RPT // _CB):
        pltpu.sync_copy(accsh.at[pl.ds(zb + j * _CB, _CB)],
                        acc_out.at[cid, pl.ds(zb + j * _CB, _CB)])
        pltpu.sync_copy(wsumsh.at[pl.ds(zb + j * _CB, _CB)],
                        wsum_out.at[cid, pl.ds(zb + j * _CB, _CB)])


_edge_msgs = pl.kernel(
    _edge_msgs_body,
    out_type=(
        jax.ShapeDtypeStruct((_NC, _NP, D), jnp.float32),
        jax.ShapeDtypeStruct((_NC, _NP, 16), jnp.float32),
    ),
    mesh=plsc.VectorSubcoreMesh(
        core_axis_name="c", subcore_axis_name="s",
        num_cores=_NC, num_subcores=_NS,
    ),
    compiler_params=pltpu.CompilerParams(needs_layout_passes=False),
    scratch_types=[
        pltpu.VMEM((_CB,), jnp.int32),
        pltpu.VMEM((_CB,), jnp.int32),
        pltpu.VMEM((_CB,), jnp.int32),
        pltpu.VMEM((_CB, D), jnp.float32),
        pltpu.VMEM((_CB, D), jnp.float32),
        pltpu.VMEM((_CB, D), jnp.float32),
        pltpu.VMEM((_CB, 16), jnp.float32),
        pltpu.VMEM_SHARED((_NP, D), jnp.float32),
        pltpu.VMEM_SHARED((_NP, 16), jnp.float32),
        pltpu.SemaphoreType.DMA,
    ],
)


def kernel(x, node_rbf, edge_sbf, params, edge_index, edge_attr, batch, edge_index_0, atom_batch):
    src = jnp.pad(edge_index[0], (0, _PAD))
    dst = jnp.pad(edge_index[1], (0, _PAD))
    attr = jnp.pad(edge_attr, (0, _PAD))
    sbf_p = jnp.pad(edge_sbf, ((0, _PAD), (0, 0)))
    valid = (jnp.arange(_E2P) < E2)[:, None]

    def readout(p, out, g_scale):
        g = out * g_scale
        per_atom = jax.ops.segment_sum(g, edge_index_0, num_segments=A)
        return _lin(p["l2"], _silu(_lin(p["l1"], per_atom)))

    out = x
    p0 = params["readout"][0]
    results = readout(p0, out, node_rbf @ p0["rbf"]["W"])

    for i in range(L):
        out_res_0 = out
        atoms_rep = jax.ops.segment_sum(out, edge_index_0, num_segments=A)
        pe = params["edgenn"][i]
        ea = _lin(pe["l2"], _silu(_lin(pe["l1"], atoms_rep)))
        pc = params["conv"][i]
        eat = _lin(pc["e"], ea)

        qx = _lin(pc["q"], out)
        kx = _lin(pc["k"], out)
        vx = _lin(pc["v"], out)

        filt = _filt_matmul(sbf_p, pc["sbf"]["W"], pc["sbf"]["b"])

        s = _edge_scores(qx, kx, eat, src, dst, attr)[:E2]
        sr, dr, ar = edge_index[0], edge_index[1], edge_attr
        m = jax.ops.segment_max(s, dr, num_segments=N)
        m = jnp.where(jnp.isfinite(m), m, 0.0)
        w8 = jnp.exp(s - m[dr])
        msg = (vx[sr] + eat[ar]).reshape(E2, H, HC)
        msg = msg * w8[:, :, None] * filt[:E2].reshape(E2, H, HC)
        acc = jax.ops.segment_sum(msg, dr, num_segments=N).reshape(N, D)
        wsum = jax.ops.segment_sum(w8, dr, num_segments=N)

        denom = (wsum + 1e-16)[:, :, None]
        out_conv = (acc.reshape(N, H, HC) / denom).reshape(N, D)
        out_conv = out_conv * (node_rbf @ pc["rbf"]["W"])

        cnt = jax.ops.segment_sum(jnp.ones((N,), jnp.float32), batch, num_segments=G) * D
        cnt = jnp.maximum(cnt, 1.0)
        mean = jax.ops.segment_sum(out_conv.sum(axis=1), batch, num_segments=G) / cnt
        xc = out_conv - mean[batch][:, None]
        var = jax.ops.segment_sum((xc * xc).sum(axis=1), batch, num_segments=G) / cnt
        out2 = xc / jnp.sqrt(var[batch][:, None] + EPS)

        pb = params["bf"][i]
        out2 = out2 + _silu(_lin(pb["l2"], _silu(_lin(pb["l1"], out2))))
        out2 = _silu(_lin(params["dense"][i], out2))
        out2 = out2 + out_res_0
        for pa in params["af"][i]:
            out2 = out2 + _silu(_lin(pa["l2"], _silu(_lin(pa["l1"], out2))))
        out = out2

        pr = params["readout"][i + 1]
        results = results + readout(pr, out, node_rbf @ pr["rbf"]["W"])

    results = jax.ops.segment_sum(results, atom_batch, num_segments=G)
    return results.reshape(-1) / L


# speedup vs baseline: 5.2454x; 5.2454x over previous
import functools

import jax
import jax.numpy as jnp
import numpy as np
from jax import lax
from jax.experimental import pallas as pl
from jax.experimental.pallas import tpu as pltpu
from jax.experimental.pallas import tpu_sc as plsc

N = 10000
E2 = 320000
A = 2500
G = 32
D = 128
RBF = 16
SBF = 112
H = 8
HC = D // H
L = 3
EPS = 1e-8

_NC = 2
_NS = 16
_NW = _NC * _NS
_C = 128
_K = 79
_EPW = _C * _K
_E2P = _NW * _EPW
_PAD = _E2P - E2
_NP = 10240
_RPT = _NP // _NS
_CB = 64
_KB = _EPW // _CB


def _lin(p, x):
    y = x @ p["W"]
    if "b" in p:
        y = y + p["b"]
    return y


def _silu(x):
    return x * jax.nn.sigmoid(x)


_FILT_BLK = 2048


def _filt_body(sbf_ref, w_ref, b_ref, o_ref):
    o_ref[...] = (
        jnp.dot(sbf_ref[...], w_ref[...], preferred_element_type=jnp.float32)
        + b_ref[...]
    )


def _filt_matmul(sbf, w, b):
    grid = (_E2P // _FILT_BLK,)
    return pl.pallas_call(
        _filt_body,
        grid=grid,
        in_specs=[
            pl.BlockSpec((_FILT_BLK, SBF), lambda i: (i, 0)),
            pl.BlockSpec((SBF, D), lambda i: (0, 0)),
            pl.BlockSpec((1, D), lambda i: (0, 0)),
        ],
        out_specs=pl.BlockSpec((_FILT_BLK, D), lambda i: (i, 0)),
        out_shape=jax.ShapeDtypeStruct((_E2P, D), jnp.float32),
    )(sbf, w, b.reshape(1, D))


def _edge_scores_body(qx, kx, eat, src, dst, attr, s_out,
                      dstb, srcb, attrb, qb, kb, eb, sb, sem):
    cid = lax.axis_index("c")
    sid = lax.axis_index("s")
    wid = sid * _NC + cid
    base0 = wid * _EPW
    iota = lax.iota(jnp.int32, 16)

    def chunk(i, carry):
        base = base0 + i * _C
        pltpu.sync_copy(dst.at[pl.ds(base, _C)], dstb)
        pltpu.sync_copy(src.at[pl.ds(base, _C)], srcb)
        pltpu.sync_copy(attr.at[pl.ds(base, _C)], attrb)
        pltpu.async_copy(qx.at[dstb], qb, sem).wait()
        pltpu.async_copy(kx.at[srcb], kb, sem).wait()
        pltpu.async_copy(eat.at[attrb], eb, sem).wait()

        def group(g, c2):
            ridx = g * 16 + iota
            for h in range(H):
                acc = jnp.zeros((16,), jnp.float32)
                for cc in range(HC):
                    col = jnp.full((16,), h * HC + cc, jnp.int32)
                    qv = plsc.load_gather(qb, [ridx, col])
                    kv = plsc.load_gather(kb, [ridx, col])
                    ev = plsc.load_gather(eb, [ridx, col])
                    acc = acc + qv * (kv + ev)
                hcol = jnp.full((16,), h, jnp.int32)
                plsc.store_scatter(sb, [ridx, hcol], acc * 0.25)
            return c2

        lax.fori_loop(0, _C // 16, group, 0)
        pltpu.sync_copy(sb, s_out.at[pl.ds(base, _C)])
        return carry

    lax.fori_loop(0, _K, chunk, 0)


_edge_scores = pl.kernel(
    _edge_scores_body,
    out_type=jax.ShapeDtypeStruct((_E2P, H), jnp.float32),
    mesh=plsc.VectorSubcoreMesh(
        core_axis_name="c", subcore_axis_name="s",
        num_cores=_NC, num_subcores=_NS,
    ),
    compiler_params=pltpu.CompilerParams(needs_layout_passes=False),
    scratch_types=[
        pltpu.VMEM((_C,), jnp.int32),
        pltpu.VMEM((_C,), jnp.int32),
        pltpu.VMEM((_C,), jnp.int32),
        pltpu.VMEM((_C, D), jnp.float32),
        pltpu.VMEM((_C, D), jnp.float32),
        pltpu.VMEM((_C, D), jnp.float32),
        pltpu.VMEM((_C, H), jnp.float32),
        pltpu.SemaphoreType.DMA,
    ],
)


def kernel(x, node_rbf, edge_sbf, params, edge_index, edge_attr, batch, edge_index_0, atom_batch):
    src = jnp.pad(edge_index[0], (0, _PAD))
    dst = jnp.pad(edge_index[1], (0, _PAD))
    attr = jnp.pad(edge_attr, (0, _PAD))
    sbf_p = jnp.pad(edge_sbf, ((0, _PAD), (0, 0)))
    valid = (jnp.arange(_E2P) < E2)[:, None]

    def readout(p, out, g_scale):
        g = out * g_scale
        per_atom = jax.ops.segment_sum(g, edge_index_0, num_segments=A)
        return _lin(p["l2"], _silu(_lin(p["l1"], per_atom)))

    out = x
    p0 = params["readout"][0]
    results = readout(p0, out, node_rbf @ p0["rbf"]["W"])

    for i in range(L):
        out_res_0 = out
        atoms_rep = jax.ops.segment_sum(out, edge_index_0, num_segments=A)
        pe = params["edgenn"][i]
        ea = _lin(pe["l2"], _silu(_lin(pe["l1"], atoms_rep)))
        pc = params["conv"][i]
        eat = _lin(pc["e"], ea)

        qx = _lin(pc["q"], out)
        kx = _lin(pc["k"], out)
        vx = _lin(pc["v"], out)

        filt = _filt_matmul(sbf_p, pc["sbf"]["W"], pc["sbf"]["b"])

        s = _edge_scores(qx, kx, eat, src, dst, attr)[:E2]
        dr = edge_index[1]
        m = jax.ops.segment_max(s, dr, num_segments=N)
        m = jnp.where(jnp.isfinite(m), m, 0.0)
        w8 = jnp.exp(s - m[dr])

        u = vx[edge_index[0]] + eat[edge_attr]
        u = (u * filt[:E2]).reshape(E2, H, HC) * w8[:, :, None]
        acc = jax.ops.segment_sum(u.reshape(E2, D), dr, num_segments=N)
        wsum = jax.ops.segment_sum(w8, dr, num_segments=N)

        denom = (wsum + 1e-16)[:, :, None]
        out_conv = (acc.reshape(N, H, HC) / denom).reshape(N, D)
        out_conv = out_conv * (node_rbf @ pc["rbf"]["W"])

        cnt = jax.ops.segment_sum(jnp.ones((N,), jnp.float32), batch, num_segments=G) * D
        cnt = jnp.maximum(cnt, 1.0)
        mean = jax.ops.segment_sum(out_conv.sum(axis=1), batch, num_segments=G) / cnt
        xc = out_conv - mean[batch][:, None]
        var = jax.ops.segment_sum((xc * xc).sum(axis=1), batch, num_segments=G) / cnt
        out2 = xc / jnp.sqrt(var[batch][:, None] + EPS)

        pb = params["bf"][i]
        out2 = out2 + _silu(_lin(pb["l2"], _silu(_lin(pb["l1"], out2))))
        out2 = _silu(_lin(params["dense"][i], out2))
        out2 = out2 + out_res_0
        for pa in params["af"][i]:
            out2 = out2 + _silu(_lin(pa["l2"], _silu(_lin(pa["l1"], out2))))
        out = out2

        pr = params["readout"][i + 1]
        results = results + readout(pr, out, node_rbf @ pr["rbf"]["W"])

    results = jax.ops.segment_sum(results, atom_batch, num_segments=G)
    return results.reshape(-1) / L
